# baseline (speedup 1.0000x reference)
import functools

import jax
import jax.numpy as jnp
from jax.experimental import pallas as pl
from jax.experimental.pallas import tpu as pltpu

VOCAB_N = 100000
D_N = 1024
B_N = 128
HIST_N = 200
NEG_F = -1.0e9

TILE_N = 2048
N_TILES = (VOCAB_N + TILE_N - 1) // TILE_N


def _matmul_kernel(h_ref, e_ref, b_ref, o_ref):
    h = h_ref[...]
    e = e_ref[...]
    o_ref[...] = jax.lax.dot_general(
        h, e, (((1,), (1,)), ((), ())), preferred_element_type=jnp.float32
    ) + b_ref[...]


def _logits_pallas(hidden_states, embedding, embedding_bias):
    bias2d = embedding_bias.reshape(1, VOCAB_N)
    return pl.pallas_call(
        _matmul_kernel,
        grid=(N_TILES,),
        in_specs=[
            pl.BlockSpec((B_N, D_N), lambda i: (0, 0)),
            pl.BlockSpec((TILE_N, D_N), lambda i: (i, 0)),
            pl.BlockSpec((1, TILE_N), lambda i: (0, i)),
        ],
        out_specs=pl.BlockSpec((B_N, TILE_N), lambda i: (0, i)),
        out_shape=jax.ShapeDtypeStruct((B_N, VOCAB_N), jnp.float32),
    )(hidden_states, embedding, bias2d)


@jax.jit
def kernel(embedding, hidden_states, embedding_bias, output_token_ids,
           presence_penalties, frequency_penalties, repetition_penalties,
           temperatures, top_ps, top_ks, min_ps):
    logits = _logits_pallas(hidden_states, embedding, embedding_bias)

    counts = jax.vmap(lambda ids: jnp.bincount(ids, length=VOCAB_N))(output_token_ids)
    countsf = counts.astype(logits.dtype)
    appeared = (counts > 0)
    rp = repetition_penalties[:, None]
    logits = jnp.where(appeared, jnp.where(logits > 0, logits / rp, logits * rp), logits)
    logits = logits - frequency_penalties[:, None] * countsf
    logits = logits - presence_penalties[:, None] * appeared.astype(logits.dtype)

    logits = logits / temperatures[:, None]

    sorted_desc = -jnp.sort(-logits, axis=-1)
    probs_sort = jax.nn.softmax(sorted_desc, axis=-1)
    cum = jnp.cumsum(probs_sort, axis=-1)
    keep_p = (cum - probs_sort) < top_ps[:, None]
    idx = jnp.arange(VOCAB_N)[None, :]
    keep_k = idx < top_ks[:, None]
    keep = keep_p & keep_k
    keep = keep.at[:, 0].set(True)
    thresh = jnp.min(jnp.where(keep, sorted_desc, jnp.inf), axis=-1, keepdims=True)
    logits = jnp.where(logits >= thresh, logits, NEG_F)

    probs = jax.nn.softmax(logits, axis=-1)
    top_prob = jnp.max(probs, axis=-1, keepdims=True)
    scaled_min_p = min_ps[:, None] * top_prob
    logits = jnp.where(probs >= scaled_min_p, logits, NEG_F)

    probs = jax.nn.softmax(logits, axis=-1)
    logprobs = jax.nn.log_softmax(logits, axis=-1)
    next_tokens = jnp.argmax(probs, axis=-1)
    sample_logprobs = jnp.take_along_axis(logprobs, next_tokens[:, None], axis=-1)[:, 0]
    return next_tokens, logprobs, sample_logprobs


# speedup vs baseline: 6.7035x; 6.7035x over previous
import functools

import jax
import jax.numpy as jnp
import numpy as np
from jax import lax
from jax.experimental import pallas as pl
from jax.experimental.pallas import tpu as pltpu
from jax.experimental.pallas import tpu_sc as plsc

VOCAB_N = 100000
D_N = 1024
B_N = 128
HIST_N = 200
NEG_F = -1.0e9

TILE_N = 2048
N_TILES = 49
VP_N = TILE_N * N_TILES

SC_NC = 2
SC_NS = 16
SC_NW = SC_NC * SC_NS
P_TOT = B_N * HIST_N
P_PER_W = P_TOT // SC_NW
P_CH = 10
P_CW = P_PER_W // P_CH

N_BISECT = 18
N_PHASES = 2 + N_BISECT + 1

_I32 = jnp.int32


def _f32_key_const(x):
    b = int(np.array(x, np.float32).view(np.int32))
    return b ^ (0x7FFFFFFF & (b >> 31)) if b < 0 else b


LO_KEY_INIT = _f32_key_const(-2.0e9)


def _key(v):
    b = lax.bitcast_convert_type(v, _I32)
    return b ^ (jnp.right_shift(b, 31) & jnp.int32(0x7FFFFFFF))


def _unkey(k):
    b = k ^ (jnp.right_shift(k, 31) & jnp.int32(0x7FFFFFFF))
    return lax.bitcast_convert_type(b, jnp.float32)


def _mid(a, b):
    return (a >> 1) + (b >> 1) + (a & b & 1)


def _matmul_body(h_ref, e_ref, b_ref, it_ref, o_ref):
    i = pl.program_id(0)
    acc = lax.dot_general(
        h_ref[...], e_ref[...], (((1,), (1,)), ((), ())),
        preferred_element_type=jnp.float32,
    )
    x = (acc + b_ref[...]) * it_ref[...]
    col = lax.broadcasted_iota(_I32, (B_N, TILE_N), 1) + i * TILE_N
    o_ref[...] = jnp.where(col < VOCAB_N, x, NEG_F)


def _logits_pallas(hidden_states, embedding, bias_pad, inv_temp):
    return pl.pallas_call(
        _matmul_body,
        grid=(N_TILES,),
        in_specs=[
            pl.BlockSpec((B_N, D_N), lambda i: (0, 0)),
            pl.BlockSpec((TILE_N, D_N), lambda i: (i, 0)),
            pl.BlockSpec((1, TILE_N), lambda i: (0, i)),
            pl.BlockSpec((B_N, 1), lambda i: (0, 0)),
        ],
        out_specs=pl.BlockSpec((B_N, TILE_N), lambda i: (0, i)),
        out_shape=jax.ShapeDtypeStruct((B_N, VP_N), jnp.float32),
    )(hidden_states, embedding, bias_pad, inv_temp)


def _penalty_body(lg_ref, idx_hbm, mp_hbm, mn_hbm, sub_hbm,
                  idx_v, val_v, mp_v, mn_v, sub_v, sem):
    wid = lax.axis_index("s") * SC_NC + lax.axis_index("c")
    pltpu.sync_copy(idx_hbm.at[wid], idx_v)
    pltpu.sync_copy(mp_hbm.at[wid], mp_v)
    pltpu.sync_copy(mn_hbm.at[wid], mn_v)
    pltpu.sync_copy(sub_hbm.at[wid], sub_v)
    for j in range(P_CH):
        pltpu.async_copy(lg_ref.at[idx_v.at[j]], val_v.at[j], sem).wait()

    def chunk(j, _):
        def vec(k, _):
            sl = pl.ds(k * 16, 16)
            x = val_v[j, sl]
            x = jnp.where(x > 0.0, x * mp_v[j, sl], x * mn_v[j, sl])
            val_v[j, sl] = x - sub_v[j, sl]
            return 0
        return lax.fori_loop(0, P_CW // 16, vec, 0)

    lax.fori_loop(0, P_CH, chunk, 0)
    for j in range(P_CH):
        pltpu.async_copy(val_v.at[j], lg_ref.at[idx_v.at[j]], sem).wait()


def _apply_penalties(logits_flat_ref, idx, mp, mn, sub):
    mesh = plsc.VectorSubcoreMesh(core_axis_name="c", subcore_axis_name="s")
    k = pl.kernel(
        _penalty_body,
        out_type=(),
        mesh=mesh,
        scratch_types=[
            pltpu.VMEM((P_CH, P_CW), _I32),
            pltpu.VMEM((P_CH, P_CW), jnp.float32),
            pltpu.VMEM((P_CH, P_CW), jnp.float32),
            pltpu.VMEM((P_CH, P_CW), jnp.float32),
            pltpu.VMEM((P_CH, P_CW), jnp.float32),
            pltpu.SemaphoreType.DMA,
        ],
    )
    k(logits_flat_ref, idx, mp, mn, sub)


def _stats_body(x_ref, tk_ref, tp_ref, mn_ref,
                m_ref, am_ref, tau_ref, mpf_ref, ks_ref,
                lo, hi, k1, k2, k3, t1, t2, t3, c1, c2, c3, s1, s2, s3, z):
    p = pl.program_id(0)
    i = pl.program_id(1)
    x = x_ref[...]

    @pl.when(p == 0)
    def _max_phase():
        tmax = jnp.max(x, axis=1, keepdims=True)
        cols = lax.broadcasted_iota(_I32, (B_N, TILE_N), 1) + i * TILE_N
        big = jnp.int32(2 ** 30)
        targ = jnp.min(jnp.where(x == tmax, cols, big), axis=1, keepdims=True)

        @pl.when(i == 0)
        def _():
            m_ref[...] = tmax
            am_ref[...] = targ

        @pl.when(i > 0)
        def _():
            upd = tmax > m_ref[...]
            am_ref[...] = jnp.where(upd, targ, am_ref[...])
            m_ref[...] = jnp.where(upd, tmax, m_ref[...])

    @pl.when(p == 1)
    def _z_phase():
        e = jnp.exp(x - m_ref[...])
        zt = jnp.sum(e, axis=1, keepdims=True)

        @pl.when(i == 0)
        def _():
            z[...] = zt

        @pl.when(i > 0)
        def _():
            z[...] = z[...] + zt

    @pl.when((p >= 2) & (i == 0))
    def _bisect_update():
        @pl.when(p == 2)
        def _():
            lo[...] = jnp.full((B_N, 1), LO_KEY_INIT, _I32)
            hi[...] = _key(m_ref[...])

        @pl.when(p > 2)
        def _():
            topk = tk_ref[...]
            pz = tp_ref[...] * z[...]
            q1 = (c1[...] + 1.0 <= topk) & (s1[...] < pz)
            q2 = (c2[...] + 1.0 <= topk) & (s2[...] < pz)
            q3 = (c3[...] + 1.0 <= topk) & (s3[...] < pz)
            lo[...] = jnp.where(
                ~q3, k3[...], jnp.where(~q2, k2[...],
                                        jnp.where(~q1, k1[...], lo[...])))
            hi[...] = jnp.where(
                q1, k1[...], jnp.where(q2, k2[...],
                                       jnp.where(q3, k3[...], hi[...])))

        @pl.when(p < N_PHASES - 1)
        def _():
            nk2 = _mid(lo[...], hi[...])
            nk1 = _mid(lo[...], nk2)
            nk3 = _mid(nk2, hi[...])
            k1[...], k2[...], k3[...] = nk1, nk2, nk3
            t1[...], t2[...], t3[...] = _unkey(nk1), _unkey(nk2), _unkey(nk3)
            zf = jnp.zeros((B_N, 1), jnp.float32)
            c1[...], c2[...], c3[...] = zf, zf, zf
            s1[...], s2[...], s3[...] = zf, zf, zf

        @pl.when(p == N_PHASES - 1)
        def _():
            tau_ref[...] = _unkey(lo[...])
            mpf_ref[...] = m_ref[...] + jnp.log(mn_ref[...])

    @pl.when((p >= 2) & (p < N_PHASES - 1))
    def _bisect_accum():
        e = jnp.exp(x - m_ref[...])
        zero = jnp.zeros((), jnp.float32)
        g1 = x > t1[...]
        g2 = x > t2[...]
        g3 = x > t3[...]
        c1[...] += jnp.sum(jnp.where(g1, 1.0, zero), axis=1, keepdims=True)
        c2[...] += jnp.sum(jnp.where(g2, 1.0, zero), axis=1, keepdims=True)
        c3[...] += jnp.sum(jnp.where(g3, 1.0, zero), axis=1, keepdims=True)
        s1[...] += jnp.sum(jnp.where(g1, e, zero), axis=1, keepdims=True)
        s2[...] += jnp.sum(jnp.where(g2, e, zero), axis=1, keepdims=True)
        s3[...] += jnp.sum(jnp.where(g3, e, zero), axis=1, keepdims=True)

    @pl.when(p == N_PHASES - 1)
    def _final_phase():
        e = jnp.exp(x - m_ref[...])
        keep = (x > tau_ref[...]) & (x >= mpf_ref[...])
        kt = jnp.sum(jnp.where(keep, e, 0.0), axis=1, keepdims=True)

        @pl.when(i == 0)
        def _():
            ks_ref[...] = kt

        @pl.when(i > 0)
        def _():
            ks_ref[...] = ks_ref[...] + kt


def _stats_pallas(logits, topk_f, topp, minp):
    small = pl.BlockSpec((B_N, 1), lambda p, i: (0, 0))
    f32 = jnp.float32
    return pl.pallas_call(
        _stats_body,
        grid=(N_PHASES, N_TILES),
        in_specs=[
            pl.BlockSpec((B_N, TILE_N), lambda p, i: (0, i)),
            small, small, small,
        ],
        out_specs=[small, small, small, small, small],
        out_shape=[
            jax.ShapeDtypeStruct((B_N, 1), f32),
            jax.ShapeDtypeStruct((B_N, 1), _I32),
            jax.ShapeDtypeStruct((B_N, 1), f32),
            jax.ShapeDtypeStruct((B_N, 1), f32),
            jax.ShapeDtypeStruct((B_N, 1), f32),
        ],
        scratch_shapes=[pltpu.VMEM((B_N, 1), _I32)] * 5
        + [pltpu.VMEM((B_N, 1), f32)] * 10,
    )(logits, topk_f, topp, minp)


def _write_body(x_ref, tau_ref, mpf_ref, lse_ref, o_ref):
    x = x_ref[...]
    keep = (x > tau_ref[...]) & (x >= mpf_ref[...])
    lse = lse_ref[...]
    o_ref[...] = jnp.where(keep, x - lse, NEG_F - lse)


def _write_pallas(logits, tau, mpf, lse):
    small = pl.BlockSpec((B_N, 1), lambda i: (0, 0))
    return pl.pallas_call(
        _write_body,
        grid=(N_TILES,),
        in_specs=[
            pl.BlockSpec((B_N, TILE_N), lambda i: (0, i)),
            small, small, small,
        ],
        out_specs=pl.BlockSpec((B_N, TILE_N), lambda i: (0, i)),
        out_shape=jax.ShapeDtypeStruct((B_N, VOCAB_N), jnp.float32),
    )(logits, tau, mpf, lse)


@jax.jit
def kernel(embedding, hidden_states, embedding_bias, output_token_ids,
           presence_penalties, frequency_penalties, repetition_penalties,
           temperatures, top_ps, top_ks, min_ps):
    f32 = jnp.float32
    inv_t = (1.0 / temperatures)[:, None]
    bias_pad = jnp.pad(embedding_bias, (0, VP_N - VOCAB_N)).reshape(1, VP_N)

    logits = _logits_pallas(hidden_states, embedding, bias_pad, inv_t)

    ids = output_token_ids
    eq = ids[:, :, None] == ids[:, None, :]
    cnt = jnp.sum(eq, axis=2).astype(f32)
    hh = jnp.arange(HIST_N, dtype=_I32)
    first = ~jnp.any(eq & (hh[None, None, :] < hh[None, :, None]), axis=2)
    rowbase = (jnp.arange(B_N, dtype=_I32) * VP_N)[:, None]
    flat = rowbase + ids
    pad_tgt = rowbase + VOCAB_N + hh[None, :]
    tgt = jnp.where(first, flat, pad_tgt)
    rp = repetition_penalties[:, None]
    mp = jnp.broadcast_to(1.0 / rp, (B_N, HIST_N))
    mn = jnp.broadcast_to(rp, (B_N, HIST_N))
    sub = (frequency_penalties[:, None] * cnt
           + presence_penalties[:, None]) * inv_t

    shp = (SC_NW, P_CH, P_CW)
    lg_ref = jax.new_ref(logits.reshape(B_N * VP_N))
    _apply_penalties(
        lg_ref,
        tgt.reshape(shp),
        mp.reshape(shp).astype(f32),
        mn.reshape(shp).astype(f32),
        sub.reshape(shp).astype(f32),
    )
    logits = lg_ref[...].reshape(B_N, VP_N)

    m, am, tau, mpf, ksum = _stats_pallas(
        logits,
        top_ks.astype(f32)[:, None],
        top_ps[:, None],
        min_ps[:, None],
    )
    lse = m + jnp.log(ksum)
    logprobs = _write_pallas(logits, tau, mpf, lse)

    next_tokens = am[:, 0]
    sample_logprobs = (m - lse)[:, 0]
    return next_tokens, logprobs, sample_logprobs
